# baseline (speedup 1.0000x reference)
import jax
import jax.numpy as jnp
from jax import lax
from jax.experimental import pallas as pl
from jax.experimental.pallas import tpu as pltpu, tpu_sc as plsc

B, L, DIM = 4096, 200, 64
VOCAB = 1001
NC, NS, LANES = 2, 16, 16
NW = NC * NS
TPW = B // NW
NFULL = L // LANES
TAIL = L - NFULL * LANES
PW = DIM // 2
NQ = DIM // LANES


def _body(table_hbm, subs_hbm, out_hbm, table_v, subs_v, out_v, sem1, sem2):
    wid = lax.axis_index("s") * NC + lax.axis_index("c")
    tok0 = wid * TPW

    cp1 = pltpu.async_copy(table_hbm, table_v, sem1)
    cp2 = pltpu.async_copy(subs_hbm.at[pl.ds(tok0, TPW)], subs_v, sem2)
    cp1.wait()
    cp2.wait()

    zero = jnp.zeros((LANES,), jnp.float32)
    izero = jnp.zeros((LANES,), jnp.int32)
    lane = lax.iota(jnp.int32, LANES)

    def chunk(idx_vec, accs, cnti):
        cnti = cnti + plsc.all_reduce_population_count(idx_vec != 0)
        bases = idx_vec * PW
        a = list(accs)
        for j in range(LANES):
            base = bases[j]
            o = (j % 2) * NQ
            pw0 = table_v[pl.ds(base, LANES)]
            pw1 = table_v[pl.ds(base + LANES, LANES)]
            a[o + 0] = a[o + 0] + plsc.bitcast(pw0 << 16, jnp.float32)
            a[o + 1] = a[o + 1] + plsc.bitcast(pw0, jnp.float32)
            a[o + 2] = a[o + 2] + plsc.bitcast(pw1 << 16, jnp.float32)
            a[o + 3] = a[o + 3] + plsc.bitcast(pw1, jnp.float32)
        return tuple(a), cnti

    def tok_body(t, _):
        @plsc.parallel_loop(0, NFULL, carry=((zero,) * (2 * NQ), izero))
        def l_result(lc, carry):
            accs, cnti = carry
            idx_vec = subs_v[t, pl.ds(lc * LANES, LANES)]
            return chunk(idx_vec, accs, cnti)

        accs, cnti = l_result

        idx_vec = subs_v[t, pl.ds(L - LANES, LANES)]
        idx_vec = jnp.where(lane >= LANES - TAIL, idx_vec, 0)
        accs, cnti = chunk(idx_vec, accs, cnti)

        cnt = cnti.astype(jnp.float32)
        for q in range(NQ):
            out_v[t, pl.ds(q * LANES, LANES)] = (accs[q] + accs[NQ + q]) / cnt
        return 0

    lax.fori_loop(0, TPW, tok_body, 0)

    pltpu.sync_copy(out_v, out_hbm.at[pl.ds(tok0, TPW)])


@jax.jit
def kernel(subs, table):
    subs = subs.astype(jnp.int32)
    table = table.astype(jnp.float32)
    u = lax.bitcast_convert_type(table.astype(jnp.bfloat16), jnp.uint16)
    u = u.astype(jnp.uint32).reshape(VOCAB, NQ, LANES)
    packed = u[:, 0::2] | (u[:, 1::2] << 16)
    packed = lax.bitcast_convert_type(packed, jnp.int32).reshape(-1)

    mesh = plsc.VectorSubcoreMesh(
        core_axis_name="c", subcore_axis_name="s", num_cores=NC, num_subcores=NS
    )
    out = pl.kernel(
        _body,
        out_type=jax.ShapeDtypeStruct((B, DIM), jnp.float32),
        mesh=mesh,
        compiler_params=pltpu.CompilerParams(needs_layout_passes=False),
        scratch_types=[
            pltpu.VMEM((VOCAB * PW,), jnp.int32),
            pltpu.VMEM((TPW, L), jnp.int32),
            pltpu.VMEM((TPW, DIM), jnp.float32),
            pltpu.SemaphoreType.DMA,
            pltpu.SemaphoreType.DMA,
        ],
    )(packed, subs)

    return out


# speedup vs baseline: 1.0250x; 1.0250x over previous
import jax
import jax.numpy as jnp
from jax import lax
from jax.experimental import pallas as pl
from jax.experimental.pallas import tpu as pltpu, tpu_sc as plsc

B, L, DIM = 4096, 200, 64
VOCAB = 1001
NC, NS, LANES = 2, 16, 16
NW = NC * NS
TPW = B // NW
NFULL = L // LANES
TAIL = L - NFULL * LANES
PW = DIM // 2
NQ = DIM // LANES


def _body(table_hbm, subs_hbm, out_hbm, table_v, subs_v, out_v, sem1, sem2):
    wid = lax.axis_index("s") * NC + lax.axis_index("c")
    tok0 = wid * TPW

    cp1 = pltpu.async_copy(table_hbm, table_v, sem1)
    cp2 = pltpu.async_copy(subs_hbm.at[pl.ds(tok0, TPW)], subs_v, sem2)
    cp1.wait()
    cp2.wait()

    zero = jnp.zeros((LANES,), jnp.float32)
    izero = jnp.zeros((LANES,), jnp.int32)
    lane = lax.iota(jnp.int32, LANES)

    def chunk(idx_vec, accs, cnti):
        cnti = cnti + plsc.all_reduce_population_count(idx_vec != 0)
        bases = idx_vec * PW
        a = list(accs)
        for j in range(LANES):
            base = bases[j]
            pw0 = table_v[pl.ds(base, LANES)]
            pw1 = table_v[pl.ds(base + LANES, LANES)]
            a[0] = a[0] + plsc.bitcast(pw0 << 16, jnp.float32)
            a[1] = a[1] + plsc.bitcast(pw0, jnp.float32)
            a[2] = a[2] + plsc.bitcast(pw1 << 16, jnp.float32)
            a[3] = a[3] + plsc.bitcast(pw1, jnp.float32)
        return tuple(a), cnti

    def tok_body(t, _):
        @plsc.parallel_loop(0, NFULL, carry=((zero,) * NQ, izero))
        def l_result(lc, carry):
            accs, cnti = carry
            idx_vec = subs_v[t, pl.ds(lc * LANES, LANES)]
            return chunk(idx_vec, accs, cnti)

        accs, cnti = l_result

        idx_vec = subs_v[t, pl.ds(L - LANES, LANES)]
        idx_vec = jnp.where(lane >= LANES - TAIL, idx_vec, 0)
        cnti = cnti + plsc.all_reduce_population_count(idx_vec != 0)
        bases = idx_vec * PW
        a = list(accs)
        for j in range(LANES - TAIL, LANES):
            base = bases[j]
            pw0 = table_v[pl.ds(base, LANES)]
            pw1 = table_v[pl.ds(base + LANES, LANES)]
            a[0] = a[0] + plsc.bitcast(pw0 << 16, jnp.float32)
            a[1] = a[1] + plsc.bitcast(pw0, jnp.float32)
            a[2] = a[2] + plsc.bitcast(pw1 << 16, jnp.float32)
            a[3] = a[3] + plsc.bitcast(pw1, jnp.float32)
        accs = tuple(a)

        rec = 1.0 / cnti.astype(jnp.float32)
        for q in range(NQ):
            out_v[t, pl.ds(q * LANES, LANES)] = accs[q] * rec
        return 0

    lax.fori_loop(0, TPW, tok_body, 0)

    pltpu.sync_copy(out_v, out_hbm.at[pl.ds(tok0, TPW)])


@jax.jit
def kernel(subs, table):
    subs = subs.astype(jnp.int32)
    table = table.astype(jnp.float32)
    u = lax.bitcast_convert_type(table.astype(jnp.bfloat16), jnp.uint16)
    u = u.astype(jnp.uint32).reshape(VOCAB, NQ, LANES)
    packed = u[:, 0::2] | (u[:, 1::2] << 16)
    packed = lax.bitcast_convert_type(packed, jnp.int32).reshape(-1)

    mesh = plsc.VectorSubcoreMesh(
        core_axis_name="c", subcore_axis_name="s", num_cores=NC, num_subcores=NS
    )
    out = pl.kernel(
        _body,
        out_type=jax.ShapeDtypeStruct((B, DIM), jnp.float32),
        mesh=mesh,
        compiler_params=pltpu.CompilerParams(needs_layout_passes=False),
        scratch_types=[
            pltpu.VMEM((VOCAB * PW,), jnp.int32),
            pltpu.VMEM((TPW, L), jnp.int32),
            pltpu.VMEM((TPW, DIM), jnp.float32),
            pltpu.SemaphoreType.DMA,
            pltpu.SemaphoreType.DMA,
        ],
    )(packed, subs)

    return out
